# Initial kernel scaffold; baseline (speedup 1.0000x reference)
import jax
import jax.numpy as jnp
from jax.experimental import pallas as pl


def kernel(x, scale, Wg, bg, W1, b1, W2, b2):
    raise NotImplementedError("write your pallas kernel here")


# speedup vs baseline: 2.7146x; 2.7146x over previous
import functools

import jax
import jax.numpy as jnp
from jax import lax
from jax.experimental import pallas as pl
from jax.experimental.pallas import tpu as pltpu
from jax.experimental.pallas import tpu_sc as plsc

T, H, FF, E, K = 2048, 768, 768, 64, 2
LIMIT, ALPHA, EPS = 7.0, 1.702, 1e-05

TM = 128
NT = T * K // TM + E
P = NT * TM
TOK_TILE = 256
NW = 32
TPW = T // NW


def _norm_gate_body(x_ref, s_ref, wg_ref, bg_ref, t_ref, g_ref):
    x = x_ref[...]
    ms = jnp.mean(x * x, axis=-1, keepdims=True)
    t = x * lax.rsqrt(ms + EPS) * s_ref[...]
    t_ref[...] = t
    g_ref[...] = jnp.dot(t, wg_ref[...], preferred_element_type=jnp.float32) + bg_ref[...]


def _norm_gate(x, scale2, Wg, bg2):
    return pl.pallas_call(
        _norm_gate_body,
        grid=(T // TOK_TILE,),
        in_specs=[
            pl.BlockSpec((TOK_TILE, H), lambda i: (i, 0)),
            pl.BlockSpec((1, H), lambda i: (0, 0)),
            pl.BlockSpec((H, E), lambda i: (0, 0)),
            pl.BlockSpec((1, E), lambda i: (0, 0)),
        ],
        out_specs=[
            pl.BlockSpec((TOK_TILE, H), lambda i: (i, 0)),
            pl.BlockSpec((TOK_TILE, E), lambda i: (i, 0)),
        ],
        out_shape=[
            jax.ShapeDtypeStruct((T, H), jnp.float32),
            jax.ShapeDtypeStruct((T, E), jnp.float32),
        ],
    )(x, scale2, Wg, bg2)


def _route_body(g_ref, r1_ref, r2_ref, w1_ref, w2_ref, te_ref, vd_ref):
    g = g_ref[...]
    ei = lax.broadcasted_iota(jnp.int32, (T, E), 1)
    v1 = jnp.max(g, axis=1, keepdims=True)
    a1 = jnp.min(jnp.where(g == v1, ei, E), axis=1, keepdims=True)
    gm = jnp.where(ei == a1, -jnp.inf, g)
    v2 = jnp.max(gm, axis=1, keepdims=True)
    a2 = jnp.min(jnp.where(gm == v2, ei, E), axis=1, keepdims=True)
    w1 = 1.0 / (1.0 + jnp.exp(v2 - v1))
    w1_ref[...] = w1[:, 0]
    w2_ref[...] = 1.0 - w1[:, 0]

    mask = jnp.where(ei == a1, 1.0, 0.0) + jnp.where(ei == a2, 1.0, 0.0)
    c = mask
    s = 1
    while s < T:
        c = c + jnp.concatenate(
            [jnp.zeros((s, E), jnp.float32), c[: T - s, :]], axis=0)
        s *= 2
    pos = c - mask
    counts = c[T - 1 : T, :]
    pci = ((counts.astype(jnp.int32) + TM - 1) // TM) * TM
    pcf = pci.astype(jnp.float32)
    lt = jnp.where(
        lax.broadcasted_iota(jnp.int32, (E, E), 0)
        < lax.broadcasted_iota(jnp.int32, (E, E), 1), 1.0, 0.0)
    off = jnp.dot(pcf, lt, preferred_element_type=jnp.float32)
    r1 = jnp.sum(jnp.where(ei == a1, off + pos, 0.0), axis=1)
    r2 = jnp.sum(jnp.where(ei == a2, off + pos, 0.0), axis=1)
    r1_ref[...] = r1.astype(jnp.int32)
    r2_ref[...] = r2.astype(jnp.int32)

    total = jnp.sum(pcf)
    offi = off.astype(jnp.int32)
    ti = lax.broadcasted_iota(jnp.int32, (NT, E), 0) * TM
    inint = (ti >= offi) & (ti < offi + pci)
    eirow = lax.broadcasted_iota(jnp.int32, (NT, E), 1).astype(jnp.float32)
    te_raw = jnp.sum(jnp.where(inint, eirow, 0.0), axis=1).astype(jnp.int32)
    vd = (lax.iota(jnp.int32, NT).astype(jnp.float32) * TM < total).astype(jnp.int32)
    te_last = jnp.max(jnp.where(vd == 1, te_raw, -1))
    te_ref[...] = jnp.where(vd == 1, te_raw, te_last)
    vd_ref[...] = vd


def _route(g):
    return pl.pallas_call(
        _route_body,
        in_specs=[pl.BlockSpec((T, E), lambda: (0, 0))],
        out_specs=[
            pl.BlockSpec((T,), lambda: (0,)),
            pl.BlockSpec((T,), lambda: (0,)),
            pl.BlockSpec((T,), lambda: (0,)),
            pl.BlockSpec((T,), lambda: (0,)),
            pl.BlockSpec((NT,), lambda: (0,)),
            pl.BlockSpec((NT,), lambda: (0,)),
        ],
        out_shape=[
            jax.ShapeDtypeStruct((T,), jnp.int32),
            jax.ShapeDtypeStruct((T,), jnp.int32),
            jax.ShapeDtypeStruct((T,), jnp.float32),
            jax.ShapeDtypeStruct((T,), jnp.float32),
            jax.ShapeDtypeStruct((NT,), jnp.int32),
            jax.ShapeDtypeStruct((NT,), jnp.int32),
        ],
    )(g)


def _dispatch_sc(t, r1, r2):
    mesh = plsc.VectorSubcoreMesh(core_axis_name="c", subcore_axis_name="s")

    @functools.partial(
        pl.kernel,
        mesh=mesh,
        out_type=jax.ShapeDtypeStruct((P, H), jnp.float32),
        scratch_types=[
            pltpu.VMEM((TPW, H), jnp.float32),
            pltpu.VMEM((TPW,), jnp.int32),
            pltpu.VMEM((TPW,), jnp.int32),
            pltpu.SemaphoreType.DMA,
        ],
    )
    def k(t_hbm, r1_hbm, r2_hbm, xg_hbm, tv, i1, i2, sem):
        wid = lax.axis_index("s") * 2 + lax.axis_index("c")
        base = wid * TPW
        pltpu.sync_copy(t_hbm.at[pl.ds(base, TPW)], tv)
        pltpu.sync_copy(r1_hbm.at[pl.ds(base, TPW)], i1)
        pltpu.sync_copy(r2_hbm.at[pl.ds(base, TPW)], i2)
        pltpu.async_copy(tv, xg_hbm.at[i1], sem).wait()
        pltpu.async_copy(tv, xg_hbm.at[i2], sem).wait()

    return k(t, r1, r2)


def _mlp_body(te_ref, vd_ref, xg_ref, w1g_ref, b1g_ref, w1l_ref, b1l_ref,
              w2_ref, b2_ref, o_ref):
    i = pl.program_id(0)

    @pl.when(vd_ref[i] == 1)
    def _():
        a = xg_ref[...]
        hg = jnp.dot(a, w1g_ref[0], preferred_element_type=jnp.float32) + b1g_ref[0]
        hl = jnp.dot(a, w1l_ref[0], preferred_element_type=jnp.float32) + b1l_ref[0]
        xglu = jnp.minimum(hg, LIMIT)
        xlin = jnp.clip(hl, -LIMIT, LIMIT)
        act = xglu * jax.nn.sigmoid(ALPHA * xglu) * (xlin + 1.0)
        o_ref[...] = jnp.dot(act, w2_ref[0], preferred_element_type=jnp.float32) + b2_ref[0]


def _mlp(te, vd, xg, W1g, b1g, W1l, b1l, W2, b2r):
    grid_spec = pltpu.PrefetchScalarGridSpec(
        num_scalar_prefetch=2,
        grid=(NT,),
        in_specs=[
            pl.BlockSpec((TM, H), lambda i, te, vd: (i, 0)),
            pl.BlockSpec((1, H, FF), lambda i, te, vd: (te[i], 0, 0)),
            pl.BlockSpec((1, 1, FF), lambda i, te, vd: (te[i], 0, 0)),
            pl.BlockSpec((1, H, FF), lambda i, te, vd: (te[i], 0, 0)),
            pl.BlockSpec((1, 1, FF), lambda i, te, vd: (te[i], 0, 0)),
            pl.BlockSpec((1, FF, H), lambda i, te, vd: (te[i], 0, 0)),
            pl.BlockSpec((1, 1, H), lambda i, te, vd: (te[i], 0, 0)),
        ],
        out_specs=pl.BlockSpec((TM, H), lambda i, te, vd: (i, 0)),
    )
    return pl.pallas_call(
        _mlp_body,
        grid_spec=grid_spec,
        out_shape=jax.ShapeDtypeStruct((P, H), jnp.float32),
    )(te, vd, xg, W1g, b1g, W1l, b1l, W2, b2r)


def _gather_sc(o, r1, r2):
    mesh = plsc.VectorSubcoreMesh(core_axis_name="c", subcore_axis_name="s")

    @functools.partial(
        pl.kernel,
        mesh=mesh,
        out_type=(
            jax.ShapeDtypeStruct((T, H), jnp.float32),
            jax.ShapeDtypeStruct((T, H), jnp.float32),
        ),
        scratch_types=[
            pltpu.VMEM((TPW, H), jnp.float32),
            pltpu.VMEM((TPW,), jnp.int32),
            pltpu.VMEM((TPW,), jnp.int32),
            pltpu.SemaphoreType.DMA,
        ],
    )
    def k(o_hbm, r1_hbm, r2_hbm, g1_hbm, g2_hbm, gv, i1, i2, sem):
        wid = lax.axis_index("s") * 2 + lax.axis_index("c")
        base = wid * TPW
        pltpu.sync_copy(r1_hbm.at[pl.ds(base, TPW)], i1)
        pltpu.sync_copy(r2_hbm.at[pl.ds(base, TPW)], i2)
        pltpu.async_copy(o_hbm.at[i1], gv, sem).wait()
        pltpu.sync_copy(gv, g1_hbm.at[pl.ds(base, TPW)])
        pltpu.async_copy(o_hbm.at[i2], gv, sem).wait()
        pltpu.sync_copy(gv, g2_hbm.at[pl.ds(base, TPW)])

    return k(o, r1, r2)


def _combine_body(x_ref, w1_ref, w2_ref, g1_ref, g2_ref, o_ref):
    o_ref[...] = (x_ref[...] + w1_ref[...] * g1_ref[...]
                  + w2_ref[...] * g2_ref[...])


def _combine(x, w1c, w2c, g1, g2):
    return pl.pallas_call(
        _combine_body,
        grid=(T // TOK_TILE,),
        in_specs=[
            pl.BlockSpec((TOK_TILE, H), lambda i: (i, 0)),
            pl.BlockSpec((TOK_TILE, 1), lambda i: (i, 0)),
            pl.BlockSpec((TOK_TILE, 1), lambda i: (i, 0)),
            pl.BlockSpec((TOK_TILE, H), lambda i: (i, 0)),
            pl.BlockSpec((TOK_TILE, H), lambda i: (i, 0)),
        ],
        out_specs=pl.BlockSpec((TOK_TILE, H), lambda i: (i, 0)),
        out_shape=jax.ShapeDtypeStruct((T, H), jnp.float32),
    )(x, w1c, w2c, g1, g2)


def kernel(x, scale, Wg, bg, W1, b1, W2, b2):
    scale2 = scale.reshape(1, H)
    bg2 = bg.reshape(1, E)
    W1g = W1[:, :, ::2]
    W1l = W1[:, :, 1::2]
    b1g = b1[:, ::2].reshape(E, 1, FF)
    b1l = b1[:, 1::2].reshape(E, 1, FF)
    b2r = b2.reshape(E, 1, H)
    t, g = _norm_gate(x, scale2, Wg, bg2)
    r1, r2, w1, w2, te, vd = _route(g)
    xg = _dispatch_sc(t, r1, r2)
    o = _mlp(te, vd, xg, W1g, b1g, W1l, b1l, W2, b2r)
    g1, g2 = _gather_sc(o, r1, r2)
    return _combine(x, w1.reshape(T, 1), w2.reshape(T, 1), g1, g2)


# speedup vs baseline: 111.4417x; 41.0522x over previous
import functools

import jax
import jax.numpy as jnp
from jax import lax
from jax.experimental import pallas as pl
from jax.experimental.pallas import tpu as pltpu
from jax.experimental.pallas import tpu_sc as plsc

T, H, FF, E, K = 2048, 768, 768, 64, 2
LIMIT, ALPHA, EPS = 7.0, 1.702, 1e-05

TM = 128
NT = T * K // TM + E
P = NT * TM
TOK_TILE = 256
NW = 32
TPW = T // NW


def _norm_gate_body(x_ref, s_ref, wg_ref, bg_ref, t_ref, g_ref):
    x = x_ref[...]
    ms = jnp.mean(x * x, axis=-1, keepdims=True)
    t = x * lax.rsqrt(ms + EPS) * s_ref[...]
    t_ref[...] = t
    g_ref[...] = jnp.dot(t, wg_ref[...], preferred_element_type=jnp.float32) + bg_ref[...]


def _norm_gate(x, scale2, Wg, bg2):
    return pl.pallas_call(
        _norm_gate_body,
        grid=(T // TOK_TILE,),
        in_specs=[
            pl.BlockSpec((TOK_TILE, H), lambda i: (i, 0)),
            pl.BlockSpec((1, H), lambda i: (0, 0)),
            pl.BlockSpec((H, E), lambda i: (0, 0)),
            pl.BlockSpec((1, E), lambda i: (0, 0)),
        ],
        out_specs=[
            pl.BlockSpec((TOK_TILE, H), lambda i: (i, 0)),
            pl.BlockSpec((TOK_TILE, E), lambda i: (i, 0)),
        ],
        out_shape=[
            jax.ShapeDtypeStruct((T, H), jnp.float32),
            jax.ShapeDtypeStruct((T, E), jnp.float32),
        ],
    )(x, scale2, Wg, bg2)


def _route_body(g_ref, r1_ref, r2_ref, w1_ref, w2_ref, te_ref, vd_ref):
    g = g_ref[...]
    ei = lax.broadcasted_iota(jnp.int32, (T, E), 1)
    v1 = jnp.max(g, axis=1, keepdims=True)
    a1 = jnp.min(jnp.where(g == v1, ei, E), axis=1, keepdims=True)
    gm = jnp.where(ei == a1, -jnp.inf, g)
    v2 = jnp.max(gm, axis=1, keepdims=True)
    a2 = jnp.min(jnp.where(gm == v2, ei, E), axis=1, keepdims=True)
    w1 = 1.0 / (1.0 + jnp.exp(v2 - v1))
    w1_ref[...] = w1[:, 0]
    w2_ref[...] = 1.0 - w1[:, 0]

    mask = jnp.where(ei == a1, 1.0, 0.0) + jnp.where(ei == a2, 1.0, 0.0)
    c = mask
    s = 1
    while s < T:
        c = c + jnp.concatenate(
            [jnp.zeros((s, E), jnp.float32), c[: T - s, :]], axis=0)
        s *= 2
    pos = c - mask
    counts = c[T - 1 : T, :]
    pci = ((counts.astype(jnp.int32) + TM - 1) // TM) * TM
    pcf = pci.astype(jnp.float32)
    lt = jnp.where(
        lax.broadcasted_iota(jnp.int32, (E, E), 0)
        < lax.broadcasted_iota(jnp.int32, (E, E), 1), 1.0, 0.0)
    off = jnp.dot(pcf, lt, preferred_element_type=jnp.float32)
    r1 = jnp.sum(jnp.where(ei == a1, off + pos, 0.0), axis=1)
    r2 = jnp.sum(jnp.where(ei == a2, off + pos, 0.0), axis=1)
    r1_ref[...] = r1.astype(jnp.int32)
    r2_ref[...] = r2.astype(jnp.int32)

    total = jnp.sum(pcf)
    offi = off.astype(jnp.int32)
    ti = lax.broadcasted_iota(jnp.int32, (NT, E), 0) * TM
    inint = (ti >= offi) & (ti < offi + pci)
    eirow = lax.broadcasted_iota(jnp.int32, (NT, E), 1).astype(jnp.float32)
    te_raw = jnp.sum(jnp.where(inint, eirow, 0.0), axis=1).astype(jnp.int32)
    vd = (lax.iota(jnp.int32, NT).astype(jnp.float32) * TM < total).astype(jnp.int32)
    te_last = jnp.max(jnp.where(vd == 1, te_raw, -1))
    te_ref[...] = jnp.where(vd == 1, te_raw, te_last)
    vd_ref[...] = vd


def _route(g):
    return pl.pallas_call(
        _route_body,
        in_specs=[pl.BlockSpec((T, E), lambda: (0, 0))],
        out_specs=[
            pl.BlockSpec((T,), lambda: (0,)),
            pl.BlockSpec((T,), lambda: (0,)),
            pl.BlockSpec((T,), lambda: (0,)),
            pl.BlockSpec((T,), lambda: (0,)),
            pl.BlockSpec((NT,), lambda: (0,)),
            pl.BlockSpec((NT,), lambda: (0,)),
        ],
        out_shape=[
            jax.ShapeDtypeStruct((T,), jnp.int32),
            jax.ShapeDtypeStruct((T,), jnp.int32),
            jax.ShapeDtypeStruct((T,), jnp.float32),
            jax.ShapeDtypeStruct((T,), jnp.float32),
            jax.ShapeDtypeStruct((NT,), jnp.int32),
            jax.ShapeDtypeStruct((NT,), jnp.int32),
        ],
    )(g)


def _dispatch_sc(t, r1, r2):
    mesh = plsc.VectorSubcoreMesh(core_axis_name="c", subcore_axis_name="s")

    @functools.partial(
        pl.kernel,
        mesh=mesh,
        out_type=jax.ShapeDtypeStruct((P, H), jnp.float32),
        scratch_types=[
            pltpu.VMEM((TPW, H), jnp.float32),
            pltpu.VMEM((TPW,), jnp.int32),
            pltpu.VMEM((TPW,), jnp.int32),
            pltpu.SemaphoreType.DMA,
        ],
    )
    def k(t_hbm, r1_hbm, r2_hbm, xg_hbm, tv, i1, i2, sem):
        wid = lax.axis_index("s") * 2 + lax.axis_index("c")
        base = wid * TPW
        pltpu.sync_copy(t_hbm.at[pl.ds(base, TPW)], tv)
        pltpu.sync_copy(r1_hbm.at[pl.ds(base, TPW)], i1)
        pltpu.sync_copy(r2_hbm.at[pl.ds(base, TPW)], i2)
        pltpu.async_copy(tv, xg_hbm.at[i1], sem).wait()
        pltpu.async_copy(tv, xg_hbm.at[i2], sem).wait()

    return k(t, r1, r2)


def _mlp_body(te_ref, vd_ref, xg_ref, w1_ref, b1_ref, w2_ref, b2_ref,
              se_ref, so_ref, o_ref):
    i = pl.program_id(0)

    @pl.when(vd_ref[i] == 1)
    def _():
        a = xg_ref[...]
        h = jnp.dot(a, w1_ref[0], preferred_element_type=jnp.float32) + b1_ref[0]
        hg = jnp.dot(h, se_ref[...], preferred_element_type=jnp.float32)
        hl = jnp.dot(h, so_ref[...], preferred_element_type=jnp.float32)
        xglu = jnp.minimum(hg, LIMIT)
        xlin = jnp.clip(hl, -LIMIT, LIMIT)
        act = xglu * jax.nn.sigmoid(ALPHA * xglu) * (xlin + 1.0)
        o_ref[...] = jnp.dot(act, w2_ref[0], preferred_element_type=jnp.float32) + b2_ref[0]


def _mlp(te, vd, xg, W1, b1r, W2, b2r, Se, So):
    grid_spec = pltpu.PrefetchScalarGridSpec(
        num_scalar_prefetch=2,
        grid=(NT,),
        in_specs=[
            pl.BlockSpec((TM, H), lambda i, te, vd: (i, 0)),
            pl.BlockSpec((1, H, 2 * FF), lambda i, te, vd: (te[i], 0, 0)),
            pl.BlockSpec((1, 1, 2 * FF), lambda i, te, vd: (te[i], 0, 0)),
            pl.BlockSpec((1, FF, H), lambda i, te, vd: (te[i], 0, 0)),
            pl.BlockSpec((1, 1, H), lambda i, te, vd: (te[i], 0, 0)),
            pl.BlockSpec((2 * FF, FF), lambda i, te, vd: (0, 0)),
            pl.BlockSpec((2 * FF, FF), lambda i, te, vd: (0, 0)),
        ],
        out_specs=pl.BlockSpec((TM, H), lambda i, te, vd: (i, 0)),
    )
    return pl.pallas_call(
        _mlp_body,
        grid_spec=grid_spec,
        out_shape=jax.ShapeDtypeStruct((P, H), jnp.float32),
    )(te, vd, xg, W1, b1r, W2, b2r, Se, So)


def _gather_sc(o, r1, r2):
    mesh = plsc.VectorSubcoreMesh(core_axis_name="c", subcore_axis_name="s")

    @functools.partial(
        pl.kernel,
        mesh=mesh,
        out_type=(
            jax.ShapeDtypeStruct((T, H), jnp.float32),
            jax.ShapeDtypeStruct((T, H), jnp.float32),
        ),
        scratch_types=[
            pltpu.VMEM((TPW, H), jnp.float32),
            pltpu.VMEM((TPW,), jnp.int32),
            pltpu.VMEM((TPW,), jnp.int32),
            pltpu.SemaphoreType.DMA,
        ],
    )
    def k(o_hbm, r1_hbm, r2_hbm, g1_hbm, g2_hbm, gv, i1, i2, sem):
        wid = lax.axis_index("s") * 2 + lax.axis_index("c")
        base = wid * TPW
        pltpu.sync_copy(r1_hbm.at[pl.ds(base, TPW)], i1)
        pltpu.sync_copy(r2_hbm.at[pl.ds(base, TPW)], i2)
        pltpu.async_copy(o_hbm.at[i1], gv, sem).wait()
        pltpu.sync_copy(gv, g1_hbm.at[pl.ds(base, TPW)])
        pltpu.async_copy(o_hbm.at[i2], gv, sem).wait()
        pltpu.sync_copy(gv, g2_hbm.at[pl.ds(base, TPW)])

    return k(o, r1, r2)


def _combine_body(x_ref, w1_ref, w2_ref, g1_ref, g2_ref, o_ref):
    o_ref[...] = (x_ref[...] + w1_ref[...] * g1_ref[...]
                  + w2_ref[...] * g2_ref[...])


def _combine(x, w1c, w2c, g1, g2):
    return pl.pallas_call(
        _combine_body,
        grid=(T // TOK_TILE,),
        in_specs=[
            pl.BlockSpec((TOK_TILE, H), lambda i: (i, 0)),
            pl.BlockSpec((TOK_TILE, 1), lambda i: (i, 0)),
            pl.BlockSpec((TOK_TILE, 1), lambda i: (i, 0)),
            pl.BlockSpec((TOK_TILE, H), lambda i: (i, 0)),
            pl.BlockSpec((TOK_TILE, H), lambda i: (i, 0)),
        ],
        out_specs=pl.BlockSpec((TOK_TILE, H), lambda i: (i, 0)),
        out_shape=jax.ShapeDtypeStruct((T, H), jnp.float32),
    )(x, w1c, w2c, g1, g2)


def kernel(x, scale, Wg, bg, W1, b1, W2, b2):
    scale2 = scale.reshape(1, H)
    bg2 = bg.reshape(1, E)
    b1r = b1.reshape(E, 1, 2 * FF)
    b2r = b2.reshape(E, 1, H)
    t, g = _norm_gate(x, scale2, Wg, bg2)
    r1, r2, w1, w2, te, vd = _route(g)
    ff2 = lax.iota(jnp.int32, 2 * FF).reshape(2 * FF, 1)
    ffc = lax.iota(jnp.int32, FF).reshape(1, FF)
    Se = (ff2 == 2 * ffc).astype(jnp.float32)
    So = (ff2 == 2 * ffc + 1).astype(jnp.float32)
    xg = _dispatch_sc(t, r1, r2)
    o = _mlp(te, vd, xg, W1, b1r, W2, b2r, Se, So)
    g1, g2 = _gather_sc(o, r1, r2)
    return _combine(x, w1.reshape(T, 1), w2.reshape(T, 1), g1, g2)


# speedup vs baseline: 113.0866x; 1.0148x over previous
import functools

import jax
import jax.numpy as jnp
from jax import lax
from jax.experimental import pallas as pl
from jax.experimental.pallas import tpu as pltpu
from jax.experimental.pallas import tpu_sc as plsc

T, H, FF, E, K = 2048, 768, 768, 64, 2
LIMIT, ALPHA, EPS = 7.0, 1.702, 1e-05

TM = 128
NT = T * K // TM + E
P = NT * TM
TOK_TILE = 256
NW = 32
TPW = T // NW


def _norm_gate_body(x_ref, s_ref, wg_ref, bg_ref, t_ref, g_ref):
    x = x_ref[...]
    ms = jnp.mean(x * x, axis=-1, keepdims=True)
    t = x * lax.rsqrt(ms + EPS) * s_ref[...]
    t_ref[...] = t
    g_ref[...] = jnp.dot(t, wg_ref[...], preferred_element_type=jnp.float32) + bg_ref[...]


def _norm_gate(x, scale2, Wg, bg2):
    return pl.pallas_call(
        _norm_gate_body,
        grid=(T // TOK_TILE,),
        in_specs=[
            pl.BlockSpec((TOK_TILE, H), lambda i: (i, 0)),
            pl.BlockSpec((1, H), lambda i: (0, 0)),
            pl.BlockSpec((H, E), lambda i: (0, 0)),
            pl.BlockSpec((1, E), lambda i: (0, 0)),
        ],
        out_specs=[
            pl.BlockSpec((TOK_TILE, H), lambda i: (i, 0)),
            pl.BlockSpec((TOK_TILE, E), lambda i: (i, 0)),
        ],
        out_shape=[
            jax.ShapeDtypeStruct((T, H), jnp.float32),
            jax.ShapeDtypeStruct((T, E), jnp.float32),
        ],
    )(x, scale2, Wg, bg2)


def _route_body(g_ref, r1_ref, r2_ref, w1_ref, w2_ref, te_ref, vd_ref):
    g = g_ref[...]
    ei = lax.broadcasted_iota(jnp.int32, (T, E), 1)
    v1 = jnp.max(g, axis=1, keepdims=True)
    a1 = jnp.min(jnp.where(g == v1, ei, E), axis=1, keepdims=True)
    gm = jnp.where(ei == a1, -jnp.inf, g)
    v2 = jnp.max(gm, axis=1, keepdims=True)
    a2 = jnp.min(jnp.where(gm == v2, ei, E), axis=1, keepdims=True)
    w1 = 1.0 / (1.0 + jnp.exp(v2 - v1))
    w1_ref[...] = w1[:, 0]
    w2_ref[...] = 1.0 - w1[:, 0]

    mask = jnp.where(ei == a1, 1.0, 0.0) + jnp.where(ei == a2, 1.0, 0.0)
    c = mask
    s = 1
    while s < T:
        c = c + jnp.concatenate(
            [jnp.zeros((s, E), jnp.float32), c[: T - s, :]], axis=0)
        s *= 2
    pos = c - mask
    counts = c[T - 1 : T, :]
    pci = ((counts.astype(jnp.int32) + TM - 1) // TM) * TM
    pcf = pci.astype(jnp.float32)
    lt = jnp.where(
        lax.broadcasted_iota(jnp.int32, (E, E), 0)
        < lax.broadcasted_iota(jnp.int32, (E, E), 1), 1.0, 0.0)
    off = jnp.dot(pcf, lt, preferred_element_type=jnp.float32)
    r1 = jnp.sum(jnp.where(ei == a1, off + pos, 0.0), axis=1)
    r2 = jnp.sum(jnp.where(ei == a2, off + pos, 0.0), axis=1)
    r1_ref[...] = r1.astype(jnp.int32)
    r2_ref[...] = r2.astype(jnp.int32)

    total = jnp.sum(pcf)
    offi = off.astype(jnp.int32)
    ti = lax.broadcasted_iota(jnp.int32, (NT, E), 0) * TM
    inint = (ti >= offi) & (ti < offi + pci)
    eirow = lax.broadcasted_iota(jnp.int32, (NT, E), 1).astype(jnp.float32)
    te_raw = jnp.sum(jnp.where(inint, eirow, 0.0), axis=1).astype(jnp.int32)
    vd = (lax.iota(jnp.int32, NT).astype(jnp.float32) * TM < total).astype(jnp.int32)
    te_last = jnp.max(jnp.where(vd == 1, te_raw, -1))
    te_ref[...] = jnp.where(vd == 1, te_raw, te_last)
    vd_ref[...] = vd


def _route(g):
    return pl.pallas_call(
        _route_body,
        in_specs=[pl.BlockSpec((T, E), lambda: (0, 0))],
        out_specs=[
            pl.BlockSpec((T,), lambda: (0,)),
            pl.BlockSpec((T,), lambda: (0,)),
            pl.BlockSpec((T,), lambda: (0,)),
            pl.BlockSpec((T,), lambda: (0,)),
            pl.BlockSpec((NT,), lambda: (0,)),
            pl.BlockSpec((NT,), lambda: (0,)),
        ],
        out_shape=[
            jax.ShapeDtypeStruct((T,), jnp.int32),
            jax.ShapeDtypeStruct((T,), jnp.int32),
            jax.ShapeDtypeStruct((T,), jnp.float32),
            jax.ShapeDtypeStruct((T,), jnp.float32),
            jax.ShapeDtypeStruct((NT,), jnp.int32),
            jax.ShapeDtypeStruct((NT,), jnp.int32),
        ],
    )(g)


def _dispatch_sc(t, r1, r2):
    mesh = plsc.VectorSubcoreMesh(core_axis_name="c", subcore_axis_name="s")

    @functools.partial(
        pl.kernel,
        mesh=mesh,
        out_type=jax.ShapeDtypeStruct((P, H), jnp.float32),
        scratch_types=[
            pltpu.VMEM((TPW, H), jnp.float32),
            pltpu.VMEM((TPW,), jnp.int32),
            pltpu.VMEM((TPW,), jnp.int32),
            pltpu.SemaphoreType.DMA,
        ],
    )
    def k(t_hbm, r1_hbm, r2_hbm, xg_hbm, tv, i1, i2, sem):
        wid = lax.axis_index("s") * 2 + lax.axis_index("c")
        base = wid * TPW
        pltpu.sync_copy(t_hbm.at[pl.ds(base, TPW)], tv)
        pltpu.sync_copy(r1_hbm.at[pl.ds(base, TPW)], i1)
        pltpu.sync_copy(r2_hbm.at[pl.ds(base, TPW)], i2)
        pltpu.async_copy(tv, xg_hbm.at[i1], sem).wait()
        pltpu.async_copy(tv, xg_hbm.at[i2], sem).wait()

    return k(t, r1, r2)


def _mlp_body(te_ref, vd_ref, xg_ref, w1_ref, b1_ref, w2_ref, b2_ref,
              se_ref, o_ref):
    i = pl.program_id(0)

    @pl.when(vd_ref[i] == 1)
    def _():
        a = xg_ref[...]
        h = jnp.dot(a, w1_ref[0], preferred_element_type=jnp.float32) + b1_ref[0]
        hn = pltpu.roll(h, 2 * FF - 1, 1)
        xglu = jnp.minimum(h, LIMIT)
        xlin = jnp.clip(hn, -LIMIT, LIMIT)
        af = xglu * jax.nn.sigmoid(ALPHA * xglu) * (xlin + 1.0)
        act = jnp.dot(af, se_ref[...], preferred_element_type=jnp.float32)
        o_ref[...] = jnp.dot(act, w2_ref[0], preferred_element_type=jnp.float32) + b2_ref[0]


def _mlp(te, vd, xg, W1, b1r, W2, b2r, Se):
    grid_spec = pltpu.PrefetchScalarGridSpec(
        num_scalar_prefetch=2,
        grid=(NT,),
        in_specs=[
            pl.BlockSpec((TM, H), lambda i, te, vd: (i, 0)),
            pl.BlockSpec((1, H, 2 * FF), lambda i, te, vd: (te[i], 0, 0)),
            pl.BlockSpec((1, 1, 2 * FF), lambda i, te, vd: (te[i], 0, 0)),
            pl.BlockSpec((1, FF, H), lambda i, te, vd: (te[i], 0, 0)),
            pl.BlockSpec((1, 1, H), lambda i, te, vd: (te[i], 0, 0)),
            pl.BlockSpec((2 * FF, FF), lambda i, te, vd: (0, 0)),
        ],
        out_specs=pl.BlockSpec((TM, H), lambda i, te, vd: (i, 0)),
    )
    return pl.pallas_call(
        _mlp_body,
        grid_spec=grid_spec,
        out_shape=jax.ShapeDtypeStruct((P, H), jnp.float32),
    )(te, vd, xg, W1, b1r, W2, b2r, Se)


def _gather_sc(o, r1, r2):
    mesh = plsc.VectorSubcoreMesh(core_axis_name="c", subcore_axis_name="s")

    @functools.partial(
        pl.kernel,
        mesh=mesh,
        out_type=(
            jax.ShapeDtypeStruct((T, H), jnp.float32),
            jax.ShapeDtypeStruct((T, H), jnp.float32),
        ),
        scratch_types=[
            pltpu.VMEM((TPW, H), jnp.float32),
            pltpu.VMEM((TPW,), jnp.int32),
            pltpu.VMEM((TPW,), jnp.int32),
            pltpu.SemaphoreType.DMA,
        ],
    )
    def k(o_hbm, r1_hbm, r2_hbm, g1_hbm, g2_hbm, gv, i1, i2, sem):
        wid = lax.axis_index("s") * 2 + lax.axis_index("c")
        base = wid * TPW
        pltpu.sync_copy(r1_hbm.at[pl.ds(base, TPW)], i1)
        pltpu.sync_copy(r2_hbm.at[pl.ds(base, TPW)], i2)
        pltpu.async_copy(o_hbm.at[i1], gv, sem).wait()
        pltpu.sync_copy(gv, g1_hbm.at[pl.ds(base, TPW)])
        pltpu.async_copy(o_hbm.at[i2], gv, sem).wait()
        pltpu.sync_copy(gv, g2_hbm.at[pl.ds(base, TPW)])

    return k(o, r1, r2)


def _combine_body(x_ref, w1_ref, w2_ref, g1_ref, g2_ref, o_ref):
    o_ref[...] = (x_ref[...] + w1_ref[...] * g1_ref[...]
                  + w2_ref[...] * g2_ref[...])


def _combine(x, w1c, w2c, g1, g2):
    return pl.pallas_call(
        _combine_body,
        grid=(T // TOK_TILE,),
        in_specs=[
            pl.BlockSpec((TOK_TILE, H), lambda i: (i, 0)),
            pl.BlockSpec((TOK_TILE, 1), lambda i: (i, 0)),
            pl.BlockSpec((TOK_TILE, 1), lambda i: (i, 0)),
            pl.BlockSpec((TOK_TILE, H), lambda i: (i, 0)),
            pl.BlockSpec((TOK_TILE, H), lambda i: (i, 0)),
        ],
        out_specs=pl.BlockSpec((TOK_TILE, H), lambda i: (i, 0)),
        out_shape=jax.ShapeDtypeStruct((T, H), jnp.float32),
    )(x, w1c, w2c, g1, g2)


def kernel(x, scale, Wg, bg, W1, b1, W2, b2):
    scale2 = scale.reshape(1, H)
    bg2 = bg.reshape(1, E)
    b1r = b1.reshape(E, 1, 2 * FF)
    b2r = b2.reshape(E, 1, H)
    t, g = _norm_gate(x, scale2, Wg, bg2)
    r1, r2, w1, w2, te, vd = _route(g)
    ff2 = lax.iota(jnp.int32, 2 * FF).reshape(2 * FF, 1)
    ffc = lax.iota(jnp.int32, FF).reshape(1, FF)
    Se = (ff2 == 2 * ffc).astype(jnp.float32)
    xg = _dispatch_sc(t, r1, r2)
    o = _mlp(te, vd, xg, W1, b1r, W2, b2r, Se)
    g1, g2 = _gather_sc(o, r1, r2)
    return _combine(x, w1.reshape(T, 1), w2.reshape(T, 1), g1, g2)


# speedup vs baseline: 115.5045x; 1.0214x over previous
import functools

import jax
import jax.numpy as jnp
from jax import lax
from jax.experimental import pallas as pl
from jax.experimental.pallas import tpu as pltpu
from jax.experimental.pallas import tpu_sc as plsc

T, H, FF, E, K = 2048, 768, 768, 64, 2
LIMIT, ALPHA, EPS = 7.0, 1.702, 1e-05

TM = 128
NT = T * K // TM + E
P = NT * TM
TOK_TILE = 256
NW = 32
TPW = T // NW


def _norm_route_body(x_ref, s_ref, wg_ref, bg_ref, t_ref,
                     r1_ref, r2_ref, w1_ref, w2_ref, te_ref, vd_ref):
    x = x_ref[...]
    ms = jnp.mean(x * x, axis=-1, keepdims=True)
    t = x * lax.rsqrt(ms + EPS) * s_ref[...]
    t_ref[...] = t
    g = jnp.dot(t, wg_ref[...], preferred_element_type=jnp.float32) + bg_ref[...]
    ei = lax.broadcasted_iota(jnp.int32, (T, E), 1)
    v1 = jnp.max(g, axis=1, keepdims=True)
    a1 = jnp.min(jnp.where(g == v1, ei, E), axis=1, keepdims=True)
    gm = jnp.where(ei == a1, -jnp.inf, g)
    v2 = jnp.max(gm, axis=1, keepdims=True)
    a2 = jnp.min(jnp.where(gm == v2, ei, E), axis=1, keepdims=True)
    w1 = 1.0 / (1.0 + jnp.exp(v2 - v1))
    w1_ref[...] = w1[:, 0]
    w2_ref[...] = 1.0 - w1[:, 0]

    mask = jnp.where(ei == a1, 1.0, 0.0) + jnp.where(ei == a2, 1.0, 0.0)
    c = mask
    s = 1
    while s < T:
        c = c + jnp.concatenate(
            [jnp.zeros((s, E), jnp.float32), c[: T - s, :]], axis=0)
        s *= 2
    pos = c - mask
    counts = c[T - 1 : T, :]
    pci = ((counts.astype(jnp.int32) + TM - 1) // TM) * TM
    pcf = pci.astype(jnp.float32)
    lt = jnp.where(
        lax.broadcasted_iota(jnp.int32, (E, E), 0)
        < lax.broadcasted_iota(jnp.int32, (E, E), 1), 1.0, 0.0)
    off = jnp.dot(pcf, lt, preferred_element_type=jnp.float32)
    r1 = jnp.sum(jnp.where(ei == a1, off + pos, 0.0), axis=1)
    r2 = jnp.sum(jnp.where(ei == a2, off + pos, 0.0), axis=1)
    r1_ref[...] = r1.astype(jnp.int32)
    r2_ref[...] = r2.astype(jnp.int32)

    total = jnp.sum(pcf)
    offi = off.astype(jnp.int32)
    ti = lax.broadcasted_iota(jnp.int32, (NT, E), 0) * TM
    inint = (ti >= offi) & (ti < offi + pci)
    eirow = lax.broadcasted_iota(jnp.int32, (NT, E), 1).astype(jnp.float32)
    te_raw = jnp.sum(jnp.where(inint, eirow, 0.0), axis=1).astype(jnp.int32)
    vd = (lax.iota(jnp.int32, NT).astype(jnp.float32) * TM < total).astype(jnp.int32)
    te_last = jnp.max(jnp.where(vd == 1, te_raw, -1))
    te_ref[...] = jnp.where(vd == 1, te_raw, te_last)
    vd_ref[...] = vd


def _norm_route(x, scale2, Wg, bg2):
    return pl.pallas_call(
        _norm_route_body,
        in_specs=[
            pl.BlockSpec((T, H), lambda: (0, 0)),
            pl.BlockSpec((1, H), lambda: (0, 0)),
            pl.BlockSpec((H, E), lambda: (0, 0)),
            pl.BlockSpec((1, E), lambda: (0, 0)),
        ],
        out_specs=[
            pl.BlockSpec((T, H), lambda: (0, 0)),
            pl.BlockSpec((T,), lambda: (0,)),
            pl.BlockSpec((T,), lambda: (0,)),
            pl.BlockSpec((T,), lambda: (0,)),
            pl.BlockSpec((T,), lambda: (0,)),
            pl.BlockSpec((NT,), lambda: (0,)),
            pl.BlockSpec((NT,), lambda: (0,)),
        ],
        out_shape=[
            jax.ShapeDtypeStruct((T, H), jnp.float32),
            jax.ShapeDtypeStruct((T,), jnp.int32),
            jax.ShapeDtypeStruct((T,), jnp.int32),
            jax.ShapeDtypeStruct((T,), jnp.float32),
            jax.ShapeDtypeStruct((T,), jnp.float32),
            jax.ShapeDtypeStruct((NT,), jnp.int32),
            jax.ShapeDtypeStruct((NT,), jnp.int32),
        ],
    )(x, scale2, Wg, bg2)


def _dispatch_sc(t, r1, r2):
    mesh = plsc.VectorSubcoreMesh(core_axis_name="c", subcore_axis_name="s")

    @functools.partial(
        pl.kernel,
        mesh=mesh,
        out_type=jax.ShapeDtypeStruct((P, H), jnp.float32),
        scratch_types=[
            pltpu.VMEM((TPW, H), jnp.float32),
            pltpu.VMEM((TPW,), jnp.int32),
            pltpu.VMEM((TPW,), jnp.int32),
            pltpu.SemaphoreType.DMA,
        ],
    )
    def k(t_hbm, r1_hbm, r2_hbm, xg_hbm, tv, i1, i2, sem):
        wid = lax.axis_index("s") * 2 + lax.axis_index("c")
        base = wid * TPW
        pltpu.sync_copy(t_hbm.at[pl.ds(base, TPW)], tv)
        pltpu.sync_copy(r1_hbm.at[pl.ds(base, TPW)], i1)
        pltpu.sync_copy(r2_hbm.at[pl.ds(base, TPW)], i2)
        pltpu.async_copy(tv, xg_hbm.at[i1], sem).wait()
        pltpu.async_copy(tv, xg_hbm.at[i2], sem).wait()

    return k(t, r1, r2)


def _mlp_body(te_ref, vd_ref, xg_ref, w1_ref, b1_ref, w2_ref, b2_ref,
              se_ref, o_ref):
    i = pl.program_id(0)

    @pl.when(vd_ref[i] == 1)
    def _():
        a = xg_ref[...]
        h = jnp.dot(a, w1_ref[0], preferred_element_type=jnp.float32) + b1_ref[0]
        hn = pltpu.roll(h, 2 * FF - 1, 1)
        xglu = jnp.minimum(h, LIMIT)
        xlin = jnp.clip(hn, -LIMIT, LIMIT)
        af = xglu * jax.nn.sigmoid(ALPHA * xglu) * (xlin + 1.0)
        act = jnp.dot(af, se_ref[...], preferred_element_type=jnp.float32)
        o_ref[...] = jnp.dot(act, w2_ref[0], preferred_element_type=jnp.float32) + b2_ref[0]


def _mlp(te, vd, xg, W1, b1r, W2, b2r, Se):
    grid_spec = pltpu.PrefetchScalarGridSpec(
        num_scalar_prefetch=2,
        grid=(NT,),
        in_specs=[
            pl.BlockSpec((TM, H), lambda i, te, vd: (i, 0)),
            pl.BlockSpec((1, H, 2 * FF), lambda i, te, vd: (te[i], 0, 0)),
            pl.BlockSpec((1, 1, 2 * FF), lambda i, te, vd: (te[i], 0, 0)),
            pl.BlockSpec((1, FF, H), lambda i, te, vd: (te[i], 0, 0)),
            pl.BlockSpec((1, 1, H), lambda i, te, vd: (te[i], 0, 0)),
            pl.BlockSpec((2 * FF, FF), lambda i, te, vd: (0, 0)),
        ],
        out_specs=pl.BlockSpec((TM, H), lambda i, te, vd: (i, 0)),
    )
    return pl.pallas_call(
        _mlp_body,
        grid_spec=grid_spec,
        out_shape=jax.ShapeDtypeStruct((P, H), jnp.float32),
    )(te, vd, xg, W1, b1r, W2, b2r, Se)


def _gather_sc(o, r1, r2):
    mesh = plsc.VectorSubcoreMesh(core_axis_name="c", subcore_axis_name="s")

    @functools.partial(
        pl.kernel,
        mesh=mesh,
        out_type=(
            jax.ShapeDtypeStruct((T, H), jnp.float32),
            jax.ShapeDtypeStruct((T, H), jnp.float32),
        ),
        scratch_types=[
            pltpu.VMEM((TPW, H), jnp.float32),
            pltpu.VMEM((TPW,), jnp.int32),
            pltpu.VMEM((TPW,), jnp.int32),
            pltpu.SemaphoreType.DMA,
        ],
    )
    def k(o_hbm, r1_hbm, r2_hbm, g1_hbm, g2_hbm, gv, i1, i2, sem):
        wid = lax.axis_index("s") * 2 + lax.axis_index("c")
        base = wid * TPW
        pltpu.sync_copy(r1_hbm.at[pl.ds(base, TPW)], i1)
        pltpu.sync_copy(r2_hbm.at[pl.ds(base, TPW)], i2)
        pltpu.async_copy(o_hbm.at[i1], gv, sem).wait()
        pltpu.sync_copy(gv, g1_hbm.at[pl.ds(base, TPW)])
        pltpu.async_copy(o_hbm.at[i2], gv, sem).wait()
        pltpu.sync_copy(gv, g2_hbm.at[pl.ds(base, TPW)])

    return k(o, r1, r2)


def _combine_body(x_ref, w1_ref, w2_ref, g1_ref, g2_ref, o_ref):
    o_ref[...] = (x_ref[...] + w1_ref[...] * g1_ref[...]
                  + w2_ref[...] * g2_ref[...])


def _combine(x, w1c, w2c, g1, g2):
    return pl.pallas_call(
        _combine_body,
        grid=(T // TOK_TILE,),
        in_specs=[
            pl.BlockSpec((TOK_TILE, H), lambda i: (i, 0)),
            pl.BlockSpec((TOK_TILE, 1), lambda i: (i, 0)),
            pl.BlockSpec((TOK_TILE, 1), lambda i: (i, 0)),
            pl.BlockSpec((TOK_TILE, H), lambda i: (i, 0)),
            pl.BlockSpec((TOK_TILE, H), lambda i: (i, 0)),
        ],
        out_specs=pl.BlockSpec((TOK_TILE, H), lambda i: (i, 0)),
        out_shape=jax.ShapeDtypeStruct((T, H), jnp.float32),
    )(x, w1c, w2c, g1, g2)


def kernel(x, scale, Wg, bg, W1, b1, W2, b2):
    scale2 = scale.reshape(1, H)
    bg2 = bg.reshape(1, E)
    b1r = b1.reshape(E, 1, 2 * FF)
    b2r = b2.reshape(E, 1, H)
    t, r1, r2, w1, w2, te, vd = _norm_route(x, scale2, Wg, bg2)
    ff2 = lax.iota(jnp.int32, 2 * FF).reshape(2 * FF, 1)
    ffc = lax.iota(jnp.int32, FF).reshape(1, FF)
    Se = (ff2 == 2 * ffc).astype(jnp.float32)
    xg = _dispatch_sc(t, r1, r2)
    o = _mlp(te, vd, xg, W1, b1r, W2, b2r, Se)
    g1, g2 = _gather_sc(o, r1, r2)
    return _combine(x, w1.reshape(T, 1), w2.reshape(T, 1), g1, g2)
